# Initial kernel scaffold; baseline (speedup 1.0000x reference)
import jax
import jax.numpy as jnp
from jax.experimental import pallas as pl


def kernel(x, edge_index, edge_attr, batch, conv1_Wl, conv1_Wr, conv1_We, conv1_att, conv1_b, bn1_g, bn1_b, skip1_W, skip1_b, conv2_Wl, conv2_Wr, conv2_We, conv2_att, conv2_b, bn2_g, bn2_b, skip2_W, skip2_b, cls_W, cls_b):
    raise NotImplementedError("write your pallas kernel here")


# speedup vs baseline: 10.9654x; 10.9654x over previous
import functools

import jax
import jax.numpy as jnp
from jax import lax
from jax.experimental import pallas as pl
from jax.experimental.pallas import tpu as pltpu
from jax.experimental.pallas import tpu_sc as plsc

_N = 10000
_E = 320000
_HID = 64
_NG = 64
_NCLS = 10
_W = 80
_BLK = 80
_NTILES = 32
_EPT = _E // _NTILES
_NBLK = _EPT // _BLK
_RPT = _N // 16
_ZR = 125


@functools.lru_cache(maxsize=None)
def _make_gat_sc():
    mesh = plsc.VectorSubcoreMesh(core_axis_name="c", subcore_axis_name="s")
    f32 = jnp.float32

    def body(src_hbm, dst_hbm, xlg_hbm, xrp_hbm, ea_hbm, att_hbm, out_hbm,
             sidx, didx, xlb, xrb, eab, sidx2, didx2, xlb2, xrb2, eab2,
             orow, zbuf, attv, tmp, table,
             sga, sgb, sgc, sga2, sgb2, sgc2):
        c = lax.axis_index("c")
        s = lax.axis_index("s")
        wid = c * 16 + s

        def zrow(i, _):
            for kk in range(_W // 16):
                zbuf[i, pl.ds(kk * 16, 16)] = jnp.zeros((16,), f32)
            return 0
        lax.fori_loop(0, _ZR, zrow, 0)
        for j in range(_RPT // _ZR):
            pltpu.sync_copy(zbuf, table.at[pl.ds(s * _RPT + j * _ZR, _ZR)])
        plsc.subcore_barrier()
        pltpu.sync_copy(att_hbm, attv)
        ebase = wid * _EPT

        def fire(b, xi, di, xb, rb, eb, sa, sb_, sc_):
            off = ebase + b * _BLK
            pltpu.sync_copy(src_hbm.at[pl.ds(off, _BLK)], xi)
            pltpu.sync_copy(dst_hbm.at[pl.ds(off, _BLK)], di)
            pltpu.async_copy(xlg_hbm.at[xi], xb, sa)
            pltpu.async_copy(xrp_hbm.at[di], rb, sb_)
            pltpu.async_copy(ea_hbm.at[pl.ds(off, _BLK)], eb, sc_)

        def drain(xi, di, xb, rb, eb, sa, sb_, sc_):
            pltpu.make_async_copy(xlg_hbm.at[xi], xb, sa).wait()
            pltpu.make_async_copy(xrp_hbm.at[di], rb, sb_).wait()
            pltpu.make_async_copy(ea_hbm.at[pl.ds(0, _BLK)], eb, sc_).wait()

        def compute(di, xb, rb, eb):
            def edge_body(e, _):
                acc = jnp.zeros((16,), f32)
                for kk in range(4):
                    sl = pl.ds(kk * 16, 16)
                    m = xb[e, sl] + rb[e, sl] + eb[e, sl]
                    m = jnp.where(m > 0.0, m, 0.2 * m)
                    acc = acc + m * attv[pl.ds(kk * 16, 16)]
                lane = lax.iota(jnp.int32, 16)
                tmp[...] = acc
                for stp in (1, 2, 4, 8):
                    a = tmp[...]
                    bsh = plsc.load_gather(tmp, [lane ^ stp])
                    tmp[...] = a + bsh
                ex = jnp.exp(tmp[...])
                for kk in range(_W // 16):
                    sl = pl.ds(kk * 16, 16)
                    orow[e, sl] = ex * xb[e, sl]
                return 0
            lax.fori_loop(0, _BLK, edge_body, 0)
            pltpu.sync_copy(orow, table.at[di], add=True)

        s0 = (sidx, didx, xlb, xrb, eab, sga, sgb, sgc)
        s1 = (sidx2, didx2, xlb2, xrb2, eab2, sga2, sgb2, sgc2)

        def slot_bufs(t):
            return t[0:5], t[5:8]

        fire(0, *s0[0:5], *s0[5:8])

        def body2(i, _):
            b0 = 2 * i
            drain(*s0[0:5], *s0[5:8])
            fire(b0 + 1, *s1[0:5], *s1[5:8])
            compute(s0[1], s0[2], s0[3], s0[4])
            drain(*s1[0:5], *s1[5:8])
            fire(b0 + 2, *s0[0:5], *s0[5:8])
            compute(s1[1], s1[2], s1[3], s1[4])
            return 0
        lax.fori_loop(0, (_NBLK - 1) // 2, body2, 0)
        drain(*s0[0:5], *s0[5:8])
        compute(s0[1], s0[2], s0[3], s0[4])
        plsc.subcore_barrier()

        roff = pl.multiple_of(s * 632, 8)

        @pl.when(s < 15)
        def _():
            pltpu.sync_copy(table.at[pl.ds(roff, 632)],
                            out_hbm.at[c, pl.ds(roff, 632)])

        @pl.when(s == 15)
        def _():
            pltpu.sync_copy(table.at[pl.ds(15 * 632, 520)],
                            out_hbm.at[c, pl.ds(15 * 632, 520)])

    scratch = [
        pltpu.VMEM((_BLK,), jnp.int32),
        pltpu.VMEM((_BLK,), jnp.int32),
        pltpu.VMEM((_BLK, _W), f32),
        pltpu.VMEM((_BLK, 64), f32),
        pltpu.VMEM((_BLK, 64), f32),
        pltpu.VMEM((_BLK,), jnp.int32),
        pltpu.VMEM((_BLK,), jnp.int32),
        pltpu.VMEM((_BLK, _W), f32),
        pltpu.VMEM((_BLK, 64), f32),
        pltpu.VMEM((_BLK, 64), f32),
        pltpu.VMEM((_BLK, _W), f32),
        pltpu.VMEM((_ZR, _W), f32),
        pltpu.VMEM((64,), f32),
        pltpu.VMEM((16,), f32),
        pltpu.VMEM_SHARED((_N, _W), f32),
    ] + [pltpu.SemaphoreType.DMA] * 6
    return pl.kernel(
        body,
        out_type=jax.ShapeDtypeStruct((2, _N, _W), f32),
        mesh=mesh,
        scratch_types=scratch,
        compiler_params=pltpu.CompilerParams(needs_layout_passes=False,
                                             use_tc_tiling_on_sc=False),
    )


def _proj1_body(xref, wref, bref, xlg0, xlg1, xrp0, xrp1, xp):
    y = jnp.dot(xref[...], wref[...], preferred_element_type=jnp.float32)
    y = y + bref[...]
    bn = y.shape[0]
    col = lax.broadcasted_iota(jnp.int32, (bn, 16), 1)
    e0 = jnp.where(col == 0, 1.0, 0.0)
    xlg0[...] = jnp.concatenate([y[:, 0:64], e0], axis=1)
    xlg1[...] = jnp.concatenate([y[:, 64:128], e0], axis=1)
    xrp0[...] = y[:, 128:192]
    xrp1[...] = y[:, 192:256]
    xp[...] = y[:, 256:384]


def _proj1(x, w, b):
    bn = 1000
    g = _N // bn
    o128 = jax.ShapeDtypeStruct((_N, 128), jnp.float32)
    return pl.pallas_call(
        _proj1_body,
        grid=(g,),
        in_specs=[
            pl.BlockSpec((bn, 128), lambda i: (i, 0)),
            pl.BlockSpec((128, 384), lambda i: (0, 0)),
            pl.BlockSpec((1, 384), lambda i: (0, 0)),
        ],
        out_specs=[pl.BlockSpec((bn, _W), lambda i: (i, 0))] * 2
                  + [pl.BlockSpec((bn, 64), lambda i: (i, 0))] * 2
                  + [pl.BlockSpec((bn, 128), lambda i: (i, 0))],
        out_shape=[jax.ShapeDtypeStruct((_N, _W), jnp.float32)] * 2
                  + [jax.ShapeDtypeStruct((_N, 64), jnp.float32)] * 2
                  + [o128],
    )(x, w, b)


def _ea_body(aref, wref, e0, e1, e2):
    y = jnp.dot(aref[...], wref[...], preferred_element_type=jnp.float32)
    e0[...] = y[:, 0:64]
    e1[...] = y[:, 64:128]
    e2[...] = y[:, 128:192]


def _ea(a, w):
    be = 4000
    g = _E // be
    o64 = jax.ShapeDtypeStruct((_E, 64), jnp.float32)
    return pl.pallas_call(
        _ea_body,
        grid=(g,),
        in_specs=[
            pl.BlockSpec((be, 16), lambda i: (i, 0)),
            pl.BlockSpec((16, 192), lambda i: (0, 0)),
        ],
        out_specs=[pl.BlockSpec((be, 64), lambda i: (i, 0))] * 3,
        out_shape=[o64] * 3,
    )(a, w)


def _make_post(nheads):
    ch = 64 * nheads

    def body(*refs):
        ts = refs[0:nheads]
        bref = refs[nheads]
        gat, psum, psumsq = refs[nheads + 1:nheads + 4]
        parts = []
        for h in range(nheads):
            num = ts[h][0, :, 0:64] + ts[h][1, :, 0:64]
            den = ts[h][0, :, 64:65] + ts[h][1, :, 64:65]
            parts.append(num / (den + 1e-16))
        g = jnp.concatenate(parts, axis=1) if nheads > 1 else parts[0]
        g = g + bref[...]
        gat[...] = g

        @pl.when(pl.program_id(0) == 0)
        def _():
            psum[...] = jnp.zeros_like(psum)
            psumsq[...] = jnp.zeros_like(psumsq)
        psum[...] += jnp.sum(g, axis=0, keepdims=True)
        psumsq[...] += jnp.sum(g * g, axis=0, keepdims=True)

    bn = 1000
    grid = (_N // bn,)

    def call(ts, b):
        return pl.pallas_call(
            body,
            grid=grid,
            in_specs=[pl.BlockSpec((2, bn, _W), lambda i: (0, i, 0))
                      for _ in range(nheads)]
                     + [pl.BlockSpec((1, ch), lambda i: (0, 0))],
            out_specs=[
                pl.BlockSpec((bn, ch), lambda i: (i, 0)),
                pl.BlockSpec((1, ch), lambda i: (0, 0)),
                pl.BlockSpec((1, ch), lambda i: (0, 0)),
            ],
            out_shape=[
                jax.ShapeDtypeStruct((_N, ch), jnp.float32),
                jax.ShapeDtypeStruct((1, ch), jnp.float32),
                jax.ShapeDtypeStruct((1, ch), jnp.float32),
            ],
        )(*ts, b)
    return call


_post_2h = _make_post(2)
_post_1h = _make_post(1)


def _proj2_body(gref, xpref, aref, bref, wref, b2ref, xlg2, xrp2, xp2):
    h = gref[...] * aref[...] + bref[...] + xpref[...]
    h = jnp.where(h > 0.0, h, jnp.exp(h) - 1.0)
    y = jnp.dot(h, wref[...], preferred_element_type=jnp.float32)
    y = y + b2ref[...]
    bn = y.shape[0]
    col = lax.broadcasted_iota(jnp.int32, (bn, 16), 1)
    e0 = jnp.where(col == 0, 1.0, 0.0)
    xlg2[...] = jnp.concatenate([y[:, 0:64], e0], axis=1)
    xrp2[...] = y[:, 64:128]
    xp2[...] = y[:, 128:192]


def _proj2(gat1, xp1, a, b, w, b2):
    bn = 1000
    g = _N // bn
    return pl.pallas_call(
        _proj2_body,
        grid=(g,),
        in_specs=[
            pl.BlockSpec((bn, 128), lambda i: (i, 0)),
            pl.BlockSpec((bn, 128), lambda i: (i, 0)),
            pl.BlockSpec((1, 128), lambda i: (0, 0)),
            pl.BlockSpec((1, 128), lambda i: (0, 0)),
            pl.BlockSpec((128, 192), lambda i: (0, 0)),
            pl.BlockSpec((1, 192), lambda i: (0, 0)),
        ],
        out_specs=[
            pl.BlockSpec((bn, _W), lambda i: (i, 0)),
            pl.BlockSpec((bn, 64), lambda i: (i, 0)),
            pl.BlockSpec((bn, 64), lambda i: (i, 0)),
        ],
        out_shape=[
            jax.ShapeDtypeStruct((_N, _W), jnp.float32),
            jax.ShapeDtypeStruct((_N, 64), jnp.float32),
            jax.ShapeDtypeStruct((_N, 64), jnp.float32),
        ],
    )(gat1, xp1, a, b, w, b2)


def _pool_body(gref, xpref, aref, bref, batref, wref, cbref, out, sums, gmax):
    i = pl.program_id(0)
    h = gref[...] * aref[...] + bref[...] + xpref[...]
    h = jnp.where(h > 0.0, h, jnp.exp(h) - 1.0)
    bb = batref[...][:, 0]
    gids = lax.broadcasted_iota(jnp.int32, (1, _NG), 1)
    onehot = (bb[:, None] == gids).astype(jnp.float32)
    hext = jnp.concatenate(
        [h, jnp.ones((h.shape[0], 1), jnp.float32)], axis=1)
    sblk = lax.dot_general(onehot, hext, (((0,), (0,)), ((), ())),
                           preferred_element_type=jnp.float32)
    neg = jnp.float32(-3e38)
    cube = jnp.where(onehot[:, :, None] > 0.0, h[:, None, :], neg)
    mblk = jnp.max(cube, axis=0)

    @pl.when(i == 0)
    def _():
        sums[...] = jnp.zeros_like(sums)
        gmax[...] = jnp.full_like(gmax, neg)
    sums[...] += sblk
    gmax[...] = jnp.maximum(gmax[...], mblk)

    @pl.when(i == pl.num_programs(0) - 1)
    def _():
        cnt = sums[:, 64:65]
        xmean = sums[:, 0:64] / jnp.maximum(cnt, 1.0)
        xmax = jnp.where(cnt > 0.0, gmax[...], 0.0)
        pooled = jnp.concatenate([xmean, xmax], axis=1)
        out[...] = jnp.dot(pooled, wref[...],
                           preferred_element_type=jnp.float32) + cbref[...]


def _pool(gat2, xp2, a, b, batcol, wc, cb):
    bp = 200
    g = _N // bp
    return pl.pallas_call(
        _pool_body,
        grid=(g,),
        in_specs=[
            pl.BlockSpec((bp, 64), lambda i: (i, 0)),
            pl.BlockSpec((bp, 64), lambda i: (i, 0)),
            pl.BlockSpec((1, 64), lambda i: (0, 0)),
            pl.BlockSpec((1, 64), lambda i: (0, 0)),
            pl.BlockSpec((bp, 1), lambda i: (i, 0)),
            pl.BlockSpec((128, _NCLS), lambda i: (0, 0)),
            pl.BlockSpec((1, _NCLS), lambda i: (0, 0)),
        ],
        out_specs=pl.BlockSpec((_NG, _NCLS), lambda i: (0, 0)),
        out_shape=jax.ShapeDtypeStruct((_NG, _NCLS), jnp.float32),
        scratch_shapes=[
            pltpu.VMEM((_NG, 65), jnp.float32),
            pltpu.VMEM((_NG, 64), jnp.float32),
        ],
    )(gat2, xp2, a, b, batcol, wc, cb)


def kernel(x, edge_index, edge_attr, batch, conv1_Wl, conv1_Wr, conv1_We,
           conv1_att, conv1_b, bn1_g, bn1_b, skip1_W, skip1_b, conv2_Wl,
           conv2_Wr, conv2_We, conv2_att, conv2_b, bn2_g, bn2_b, skip2_W,
           skip2_b, cls_W, cls_b):
    src = edge_index[0]
    dst = edge_index[1]

    wc1 = jnp.concatenate([conv1_Wl.T, conv1_Wr.T, skip1_W.T], axis=1)
    bc1 = jnp.concatenate([jnp.zeros((256,), jnp.float32), skip1_b])[None, :]
    xlg0, xlg1, xrp0, xrp1, xp1 = _proj1(x, wc1, bc1)

    wec = jnp.concatenate([conv1_We.T, conv2_We.T], axis=1)
    ea0, ea1, ea2 = _ea(edge_attr, wec)

    gat_sc = _make_gat_sc()
    t10 = gat_sc(src, dst, xlg0, xrp0, ea0, conv1_att[0])
    t11 = gat_sc(src, dst, xlg1, xrp1, ea1, conv1_att[1])
    gat1, s1, ss1 = _post_2h([t10, t11], conv1_b[None, :])
    mean1 = s1 / _N
    var1 = ss1 / _N - mean1 * mean1
    rstd1 = 1.0 / jnp.sqrt(var1 + 1e-5)
    a1 = rstd1 * bn1_g[None, :]
    b1 = bn1_b[None, :] - mean1 * a1

    wc2 = jnp.concatenate([conv2_Wl.T, conv2_Wr.T, skip2_W.T], axis=1)
    bc2 = jnp.concatenate([jnp.zeros((128,), jnp.float32), skip2_b])[None, :]
    xlg2, xrp2, xp2 = _proj2(gat1, xp1, a1, b1, wc2, bc2)

    t2 = gat_sc(src, dst, xlg2, xrp2, ea2, conv2_att[0])
    gat2, s2, ss2 = _post_1h([t2], conv2_b[None, :])
    mean2 = s2 / _N
    var2 = ss2 / _N - mean2 * mean2
    rstd2 = 1.0 / jnp.sqrt(var2 + 1e-5)
    a2 = rstd2 * bn2_g[None, :]
    b2 = bn2_b[None, :] - mean2 * a2

    return _pool(gat2, xp2, a2, b2, batch.reshape(-1, 1), cls_W.T,
                 cls_b[None, :])


# speedup vs baseline: 11.5569x; 1.0539x over previous
import functools

import jax
import jax.numpy as jnp
from jax import lax
from jax.experimental import pallas as pl
from jax.experimental.pallas import tpu as pltpu
from jax.experimental.pallas import tpu_sc as plsc

_N = 10000
_E = 320000
_HID = 64
_NG = 64
_NCLS = 10
_W = 80
_BLK = 80
_NTILES = 32
_EPT = _E // _NTILES
_NBLK = _EPT // _BLK
_RPT = _N // 16
_ZR = 125


@functools.lru_cache(maxsize=None)
def _make_gat_sc():
    mesh = plsc.VectorSubcoreMesh(core_axis_name="c", subcore_axis_name="s")
    f32 = jnp.float32

    def body(src_hbm, dst_hbm, xlg_hbm, xrp_hbm, ea_hbm, att_hbm, out_hbm,
             sidx, didx, xlb, xrb, eab, sidx2, didx2, xlb2, xrb2, eab2,
             orow, zbuf, attv, tmp, table,
             sga, sgb, sgc, sga2, sgb2, sgc2):
        c = lax.axis_index("c")
        s = lax.axis_index("s")
        wid = c * 16 + s

        def zrow(i, _):
            for kk in range(_W // 16):
                zbuf[i, pl.ds(kk * 16, 16)] = jnp.zeros((16,), f32)
            return 0
        lax.fori_loop(0, _ZR, zrow, 0)
        for j in range(_RPT // _ZR):
            pltpu.sync_copy(zbuf, table.at[pl.ds(s * _RPT + j * _ZR, _ZR)])
        plsc.subcore_barrier()
        pltpu.sync_copy(att_hbm, attv)
        ebase = wid * _EPT

        def fire(b, xi, di, xb, rb, eb, sa, sb_, sc_):
            off = ebase + b * _BLK
            pltpu.sync_copy(src_hbm.at[pl.ds(off, _BLK)], xi)
            pltpu.sync_copy(dst_hbm.at[pl.ds(off, _BLK)], di)
            pltpu.async_copy(xlg_hbm.at[xi], xb, sa)
            pltpu.async_copy(xrp_hbm.at[di], rb, sb_)
            pltpu.async_copy(ea_hbm.at[pl.ds(off, _BLK)], eb, sc_)

        def drain(xi, di, xb, rb, eb, sa, sb_, sc_):
            pltpu.make_async_copy(xlg_hbm.at[xi], xb, sa).wait()
            pltpu.make_async_copy(xrp_hbm.at[di], rb, sb_).wait()
            pltpu.make_async_copy(ea_hbm.at[pl.ds(0, _BLK)], eb, sc_).wait()

        def compute(di, xb, rb, eb):
            lane = lax.iota(jnp.int32, 16)
            e0c = jnp.where(lane == 0, 1.0, 0.0).astype(f32)

            def edge_body(e, _):
                acc = jnp.zeros((16,), f32)
                for kk in range(4):
                    sl = pl.ds(kk * 16, 16)
                    m = xb[e, sl] + rb[e, sl] + eb[e, sl]
                    m = jnp.where(m > 0.0, m, 0.2 * m)
                    acc = acc + m * attv[pl.ds(kk * 16, 16)]
                tmp[...] = acc
                for stp in (1, 2, 4, 8):
                    a = tmp[...]
                    bsh = plsc.load_gather(tmp, [lane ^ stp])
                    tmp[...] = a + bsh
                ex = jnp.exp(tmp[...])
                for kk in range(4):
                    sl = pl.ds(kk * 16, 16)
                    orow[e, sl] = ex * xb[e, sl]
                orow[e, pl.ds(64, 16)] = ex * e0c
                return 0
            lax.fori_loop(0, _BLK, edge_body, 0)
            pltpu.sync_copy(orow, table.at[di], add=True)

        s0 = (sidx, didx, xlb, xrb, eab, sga, sgb, sgc)
        s1 = (sidx2, didx2, xlb2, xrb2, eab2, sga2, sgb2, sgc2)

        def slot_bufs(t):
            return t[0:5], t[5:8]

        fire(0, *s0[0:5], *s0[5:8])

        def body2(i, _):
            b0 = 2 * i
            drain(*s0[0:5], *s0[5:8])
            fire(b0 + 1, *s1[0:5], *s1[5:8])
            compute(s0[1], s0[2], s0[3], s0[4])
            drain(*s1[0:5], *s1[5:8])
            fire(b0 + 2, *s0[0:5], *s0[5:8])
            compute(s1[1], s1[2], s1[3], s1[4])
            return 0
        lax.fori_loop(0, (_NBLK - 1) // 2, body2, 0)
        drain(*s0[0:5], *s0[5:8])
        compute(s0[1], s0[2], s0[3], s0[4])
        plsc.subcore_barrier()

        roff = pl.multiple_of(s * 632, 8)

        @pl.when(s < 15)
        def _():
            pltpu.sync_copy(table.at[pl.ds(roff, 632)],
                            out_hbm.at[c, pl.ds(roff, 632)])

        @pl.when(s == 15)
        def _():
            pltpu.sync_copy(table.at[pl.ds(15 * 632, 520)],
                            out_hbm.at[c, pl.ds(15 * 632, 520)])

    scratch = [
        pltpu.VMEM((_BLK,), jnp.int32),
        pltpu.VMEM((_BLK,), jnp.int32),
        pltpu.VMEM((_BLK, 64), f32),
        pltpu.VMEM((_BLK, 64), f32),
        pltpu.VMEM((_BLK, 64), f32),
        pltpu.VMEM((_BLK,), jnp.int32),
        pltpu.VMEM((_BLK,), jnp.int32),
        pltpu.VMEM((_BLK, 64), f32),
        pltpu.VMEM((_BLK, 64), f32),
        pltpu.VMEM((_BLK, 64), f32),
        pltpu.VMEM((_BLK, _W), f32),
        pltpu.VMEM((_ZR, _W), f32),
        pltpu.VMEM((64,), f32),
        pltpu.VMEM((16,), f32),
        pltpu.VMEM_SHARED((_N, _W), f32),
    ] + [pltpu.SemaphoreType.DMA] * 6
    return pl.kernel(
        body,
        out_type=jax.ShapeDtypeStruct((2, _N, _W), f32),
        mesh=mesh,
        scratch_types=scratch,
        compiler_params=pltpu.CompilerParams(needs_layout_passes=False,
                                             use_tc_tiling_on_sc=False),
    )


def _proj1_body(xref, wref, bref, xlg0, xlg1, xrp0, xrp1, xp):
    y = jnp.dot(xref[...], wref[...], preferred_element_type=jnp.float32)
    y = y + bref[...]
    xlg0[...] = y[:, 0:64]
    xlg1[...] = y[:, 64:128]
    xrp0[...] = y[:, 128:192]
    xrp1[...] = y[:, 192:256]
    xp[...] = y[:, 256:384]


def _proj1(x, w, b):
    bn = 1000
    g = _N // bn
    o128 = jax.ShapeDtypeStruct((_N, 128), jnp.float32)
    return pl.pallas_call(
        _proj1_body,
        grid=(g,),
        in_specs=[
            pl.BlockSpec((bn, 128), lambda i: (i, 0)),
            pl.BlockSpec((128, 384), lambda i: (0, 0)),
            pl.BlockSpec((1, 384), lambda i: (0, 0)),
        ],
        out_specs=[pl.BlockSpec((bn, 64), lambda i: (i, 0))] * 4
                  + [pl.BlockSpec((bn, 128), lambda i: (i, 0))],
        out_shape=[jax.ShapeDtypeStruct((_N, 64), jnp.float32)] * 4
                  + [o128],
    )(x, w, b)


def _ea_body(aref, wref, e0, e1, e2):
    y = jnp.dot(aref[...], wref[...], preferred_element_type=jnp.float32)
    e0[...] = y[:, 0:64]
    e1[...] = y[:, 64:128]
    e2[...] = y[:, 128:192]


def _ea(a, w):
    be = 4000
    g = _E // be
    o64 = jax.ShapeDtypeStruct((_E, 64), jnp.float32)
    return pl.pallas_call(
        _ea_body,
        grid=(g,),
        in_specs=[
            pl.BlockSpec((be, 16), lambda i: (i, 0)),
            pl.BlockSpec((16, 192), lambda i: (0, 0)),
        ],
        out_specs=[pl.BlockSpec((be, 64), lambda i: (i, 0))] * 3,
        out_shape=[o64] * 3,
    )(a, w)


def _make_post(nheads):
    ch = 64 * nheads

    def body(*refs):
        ts = refs[0:nheads]
        bref = refs[nheads]
        gat, psum, psumsq = refs[nheads + 1:nheads + 4]
        parts = []
        for h in range(nheads):
            num = ts[h][0, :, 0:64] + ts[h][1, :, 0:64]
            den = ts[h][0, :, 64:65] + ts[h][1, :, 64:65]
            parts.append(num / (den + 1e-16))
        g = jnp.concatenate(parts, axis=1) if nheads > 1 else parts[0]
        g = g + bref[...]
        gat[...] = g

        @pl.when(pl.program_id(0) == 0)
        def _():
            psum[...] = jnp.zeros_like(psum)
            psumsq[...] = jnp.zeros_like(psumsq)
        psum[...] += jnp.sum(g, axis=0, keepdims=True)
        psumsq[...] += jnp.sum(g * g, axis=0, keepdims=True)

    bn = 1000
    grid = (_N // bn,)

    def call(ts, b):
        return pl.pallas_call(
            body,
            grid=grid,
            in_specs=[pl.BlockSpec((2, bn, _W), lambda i: (0, i, 0))
                      for _ in range(nheads)]
                     + [pl.BlockSpec((1, ch), lambda i: (0, 0))],
            out_specs=[
                pl.BlockSpec((bn, ch), lambda i: (i, 0)),
                pl.BlockSpec((1, ch), lambda i: (0, 0)),
                pl.BlockSpec((1, ch), lambda i: (0, 0)),
            ],
            out_shape=[
                jax.ShapeDtypeStruct((_N, ch), jnp.float32),
                jax.ShapeDtypeStruct((1, ch), jnp.float32),
                jax.ShapeDtypeStruct((1, ch), jnp.float32),
            ],
        )(*ts, b)
    return call


_post_2h = _make_post(2)
_post_1h = _make_post(1)


def _proj2_body(gref, xpref, aref, bref, wref, b2ref, xlg2, xrp2, xp2):
    h = gref[...] * aref[...] + bref[...] + xpref[...]
    h = jnp.where(h > 0.0, h, jnp.exp(h) - 1.0)
    y = jnp.dot(h, wref[...], preferred_element_type=jnp.float32)
    y = y + b2ref[...]
    xlg2[...] = y[:, 0:64]
    xrp2[...] = y[:, 64:128]
    xp2[...] = y[:, 128:192]


def _proj2(gat1, xp1, a, b, w, b2):
    bn = 1000
    g = _N // bn
    return pl.pallas_call(
        _proj2_body,
        grid=(g,),
        in_specs=[
            pl.BlockSpec((bn, 128), lambda i: (i, 0)),
            pl.BlockSpec((bn, 128), lambda i: (i, 0)),
            pl.BlockSpec((1, 128), lambda i: (0, 0)),
            pl.BlockSpec((1, 128), lambda i: (0, 0)),
            pl.BlockSpec((128, 192), lambda i: (0, 0)),
            pl.BlockSpec((1, 192), lambda i: (0, 0)),
        ],
        out_specs=[
            pl.BlockSpec((bn, 64), lambda i: (i, 0)),
            pl.BlockSpec((bn, 64), lambda i: (i, 0)),
            pl.BlockSpec((bn, 64), lambda i: (i, 0)),
        ],
        out_shape=[
            jax.ShapeDtypeStruct((_N, 64), jnp.float32),
            jax.ShapeDtypeStruct((_N, 64), jnp.float32),
            jax.ShapeDtypeStruct((_N, 64), jnp.float32),
        ],
    )(gat1, xp1, a, b, w, b2)


def _pool_body(gref, xpref, aref, bref, batref, wref, cbref, out, sums, gmax):
    i = pl.program_id(0)
    h = gref[...] * aref[...] + bref[...] + xpref[...]
    h = jnp.where(h > 0.0, h, jnp.exp(h) - 1.0)
    bb = batref[...][:, 0]
    gids = lax.broadcasted_iota(jnp.int32, (1, _NG), 1)
    onehot = (bb[:, None] == gids).astype(jnp.float32)
    hext = jnp.concatenate(
        [h, jnp.ones((h.shape[0], 1), jnp.float32)], axis=1)
    sblk = lax.dot_general(onehot, hext, (((0,), (0,)), ((), ())),
                           preferred_element_type=jnp.float32)
    neg = jnp.float32(-3e38)
    cube = jnp.where(onehot[:, :, None] > 0.0, h[:, None, :], neg)
    mblk = jnp.max(cube, axis=0)

    @pl.when(i == 0)
    def _():
        sums[...] = jnp.zeros_like(sums)
        gmax[...] = jnp.full_like(gmax, neg)
    sums[...] += sblk
    gmax[...] = jnp.maximum(gmax[...], mblk)

    @pl.when(i == pl.num_programs(0) - 1)
    def _():
        cnt = sums[:, 64:65]
        xmean = sums[:, 0:64] / jnp.maximum(cnt, 1.0)
        xmax = jnp.where(cnt > 0.0, gmax[...], 0.0)
        pooled = jnp.concatenate([xmean, xmax], axis=1)
        out[...] = jnp.dot(pooled, wref[...],
                           preferred_element_type=jnp.float32) + cbref[...]


def _pool(gat2, xp2, a, b, batcol, wc, cb):
    bp = 200
    g = _N // bp
    return pl.pallas_call(
        _pool_body,
        grid=(g,),
        in_specs=[
            pl.BlockSpec((bp, 64), lambda i: (i, 0)),
            pl.BlockSpec((bp, 64), lambda i: (i, 0)),
            pl.BlockSpec((1, 64), lambda i: (0, 0)),
            pl.BlockSpec((1, 64), lambda i: (0, 0)),
            pl.BlockSpec((bp, 1), lambda i: (i, 0)),
            pl.BlockSpec((128, _NCLS), lambda i: (0, 0)),
            pl.BlockSpec((1, _NCLS), lambda i: (0, 0)),
        ],
        out_specs=pl.BlockSpec((_NG, _NCLS), lambda i: (0, 0)),
        out_shape=jax.ShapeDtypeStruct((_NG, _NCLS), jnp.float32),
        scratch_shapes=[
            pltpu.VMEM((_NG, 65), jnp.float32),
            pltpu.VMEM((_NG, 64), jnp.float32),
        ],
    )(gat2, xp2, a, b, batcol, wc, cb)


def kernel(x, edge_index, edge_attr, batch, conv1_Wl, conv1_Wr, conv1_We,
           conv1_att, conv1_b, bn1_g, bn1_b, skip1_W, skip1_b, conv2_Wl,
           conv2_Wr, conv2_We, conv2_att, conv2_b, bn2_g, bn2_b, skip2_W,
           skip2_b, cls_W, cls_b):
    src = edge_index[0]
    dst = edge_index[1]

    wc1 = jnp.concatenate([conv1_Wl.T, conv1_Wr.T, skip1_W.T], axis=1)
    bc1 = jnp.concatenate([jnp.zeros((256,), jnp.float32), skip1_b])[None, :]
    xlg0, xlg1, xrp0, xrp1, xp1 = _proj1(x, wc1, bc1)

    wec = jnp.concatenate([conv1_We.T, conv2_We.T], axis=1)
    ea0, ea1, ea2 = _ea(edge_attr, wec)

    gat_sc = _make_gat_sc()
    t10 = gat_sc(src, dst, xlg0, xrp0, ea0, conv1_att[0])
    t11 = gat_sc(src, dst, xlg1, xrp1, ea1, conv1_att[1])
    gat1, s1, ss1 = _post_2h([t10, t11], conv1_b[None, :])
    mean1 = s1 / _N
    var1 = ss1 / _N - mean1 * mean1
    rstd1 = 1.0 / jnp.sqrt(var1 + 1e-5)
    a1 = rstd1 * bn1_g[None, :]
    b1 = bn1_b[None, :] - mean1 * a1

    wc2 = jnp.concatenate([conv2_Wl.T, conv2_Wr.T, skip2_W.T], axis=1)
    bc2 = jnp.concatenate([jnp.zeros((128,), jnp.float32), skip2_b])[None, :]
    xlg2, xrp2, xp2 = _proj2(gat1, xp1, a1, b1, wc2, bc2)

    t2 = gat_sc(src, dst, xlg2, xrp2, ea2, conv2_att[0])
    gat2, s2, ss2 = _post_1h([t2], conv2_b[None, :])
    mean2 = s2 / _N
    var2 = ss2 / _N - mean2 * mean2
    rstd2 = 1.0 / jnp.sqrt(var2 + 1e-5)
    a2 = rstd2 * bn2_g[None, :]
    b2 = bn2_b[None, :] - mean2 * a2

    return _pool(gat2, xp2, a2, b2, batch.reshape(-1, 1), cls_W.T,
                 cls_b[None, :])
